# baseline (speedup 1.0000x reference)
import functools

import jax
import jax.numpy as jnp
from jax import lax
from jax.experimental import pallas as pl
from jax.experimental.pallas import tpu as pltpu
from jax.experimental.pallas import tpu_sc as plsc

_VOCAB = 1000000
_D = 64
_BATCH = 4096
_HIST = 200

_NC, _NS = 2, 16
_NW = _NC * _NS
_RPW = _BATCH // _NW
_KA = 128
_KB = _HIST - _KA

_mesh = plsc.VectorSubcoreMesh(
    core_axis_name="c", subcore_axis_name="s",
    num_cores=_NC, num_subcores=_NS)


@functools.partial(
    pl.kernel,
    out_type=jax.ShapeDtypeStruct((_BATCH, _HIST, _D), jnp.float32),
    mesh=_mesh,
    scratch_types=[
        pltpu.VMEM((_RPW, _KA), jnp.int32),
        pltpu.VMEM((_RPW, _KB), jnp.int32),
        pltpu.VMEM((2, _KA, _D), jnp.float32),
        pltpu.VMEM((2, _KB, _D), jnp.float32),
    ] + [pltpu.SemaphoreType.DMA] * 4,
    compiler_params=pltpu.CompilerParams(use_tc_tiling_on_sc=False),
)
def _emb_gather(x_hbm, table_hbm, out_hbm, idx_a, idx_b, buf_a, buf_b,
                sa0, sa1, sb0, sb1):
    sems_a = (sa0, sa1)
    sems_b = (sb0, sb1)
    wid = lax.axis_index("s") * _NC + lax.axis_index("c")
    row0 = wid * _RPW

    pltpu.sync_copy(x_hbm.at[pl.ds(row0, _RPW), pl.ds(0, _KA)], idx_a)
    pltpu.sync_copy(x_hbm.at[pl.ds(row0, _RPW), pl.ds(_KA, _KB)], idx_b)

    def desc_a(r, p):
        return pltpu.make_async_copy(
            table_hbm.at[idx_a.at[r]], buf_a.at[p], sems_a[p])

    def desc_b(r, p):
        return pltpu.make_async_copy(
            table_hbm.at[idx_b.at[r]], buf_b.at[p], sems_b[p])

    for p in range(2):
        desc_a(p, p).start()
        desc_b(p, p).start()

    def group(g, carry):
        for p in range(2):
            r = g * 2 + p
            desc_a(r, p).wait()
            pltpu.sync_copy(buf_a.at[p], out_hbm.at[row0 + r, pl.ds(0, _KA)])
            desc_b(r, p).wait()
            pltpu.sync_copy(buf_b.at[p], out_hbm.at[row0 + r, pl.ds(_KA, _KB)])

            @pl.when(r + 2 < _RPW)
            def _():
                desc_a(r + 2, p).start()
                desc_b(r + 2, p).start()
        return carry

    lax.fori_loop(0, _RPW // 2, group, 0)


def kernel(x, table):
    return _emb_gather(x.astype(jnp.int32), table)
